# baseline (speedup 1.0000x reference)
import functools

import jax
import jax.numpy as jnp
from jax import lax
from jax.experimental import pallas as pl
from jax.experimental.pallas import tpu as pltpu
from jax.experimental.pallas import tpu_sc as plsc

EMBED = 64
BATCH = 16384
NC = 2
NS = 16
NW = NC * NS
BPW = BATCH // NW
CHUNK = 128
NCH = BPW // CHUNK
LANES = 16
EC = EMBED // LANES


def _sc_body(tbl, tgt, ctx, w, b, out, idx_t, idx_c, rows_t, rows_c,
             out_v, w_v, b_v, sems):
    wid = lax.axis_index("s") * NC + lax.axis_index("c")
    base = wid * BPW

    idx_copies = []
    for j in range(NCH):
        idx_copies.append(
            (pltpu.async_copy(tgt.at[pl.ds(base + j * CHUNK, CHUNK)],
                              idx_t.at[j], sems[j]),
             pltpu.async_copy(ctx.at[pl.ds(base + j * CHUNK, CHUNK)],
                              idx_c.at[j], sems[j])))
    pltpu.sync_copy(w, w_v)
    pltpu.sync_copy(b, b_v)

    copies = []
    for j in range(NCH):
        idx_copies[j][0].wait()
        idx_copies[j][1].wait()
        copies.append(
            (pltpu.async_copy(tbl.at[idx_t.at[j]], rows_t.at[j], sems[j]),
             pltpu.async_copy(tbl.at[idx_c.at[j]], rows_c.at[j], sems[j])))

    wregs = [w_v[pl.ds(k * LANES, LANES)] for k in range(EC)]
    bias = b_v[...][0]
    lanes = lax.iota(jnp.int32, LANES)

    for j in range(NCH):
        copies[j][0].wait()
        copies[j][1].wait()

        def group(g, _, j=j):
            out_reg = jnp.zeros((LANES,), jnp.float32)
            for k in range(LANES):
                row = g * LANES + k
                acc = None
                for e in range(EC):
                    t = rows_t[j, row, pl.ds(e * LANES, LANES)]
                    c = rows_c[j, row, pl.ds(e * LANES, LANES)]
                    term = (t * c) * wregs[e]
                    acc = term if acc is None else acc + term
                s = jnp.sum(acc)
                out_reg = jnp.where(lanes == k, s, out_reg)
            out_v[pl.ds(j * CHUNK + g * LANES, LANES)] = out_reg + bias
            return _

        lax.fori_loop(0, CHUNK // LANES, group, None)

    pltpu.sync_copy(out_v, out.at[pl.ds(base, BPW)])


@functools.partial(
    pl.kernel,
    mesh=plsc.VectorSubcoreMesh(core_axis_name="c", subcore_axis_name="s"),
    out_type=jax.ShapeDtypeStruct((BATCH,), jnp.float32),
    scratch_types=[
        pltpu.VMEM((NCH, CHUNK), jnp.int32),
        pltpu.VMEM((NCH, CHUNK), jnp.int32),
        pltpu.VMEM((NCH, CHUNK, EMBED), jnp.float32),
        pltpu.VMEM((NCH, CHUNK, EMBED), jnp.float32),
        pltpu.VMEM((BPW,), jnp.float32),
        pltpu.VMEM((EMBED,), jnp.float32),
        pltpu.VMEM((16,), jnp.float32),
        [pltpu.SemaphoreType.DMA] * NCH,
    ],
    compiler_params=pltpu.CompilerParams(
        needs_layout_passes=False, use_tc_tiling_on_sc=False),
)
def _sc_kernel(tbl, tgt, ctx, w, b, out, idx_t, idx_c, rows_t, rows_c,
               out_v, w_v, b_v, sem):
    _sc_body(tbl, tgt, ctx, w, b, out, idx_t, idx_c, rows_t, rows_c,
             out_v, w_v, b_v, sem)


def kernel(target, context, embed_table, cls_w, cls_b):
    w = cls_w.reshape((EMBED,))
    b = jnp.broadcast_to(cls_b.reshape(()), (16,))
    return _sc_kernel(embed_table, target.astype(jnp.int32),
                      context.astype(jnp.int32), w, b)
